# baseline (speedup 1.0000x reference)
import jax
import jax.numpy as jnp
from jax.experimental import pallas as pl
from jax.experimental.pallas import tpu as pltpu


def _embed_kernel(src_hbm, out_ref, sem):
    copy = pltpu.make_async_copy(src_hbm, out_ref.at[0], sem)
    copy.start()
    copy.wait()


def kernel(x, pos_table):
    maxlen = x.shape[-1]
    embed_dim = pos_table.shape[-1]
    return pl.pallas_call(
        _embed_kernel,
        in_specs=[pl.BlockSpec(memory_space=pl.ANY)],
        out_specs=pl.BlockSpec((1, maxlen, embed_dim), lambda: (0, 0, 0)),
        out_shape=jax.ShapeDtypeStruct((1, maxlen, embed_dim), pos_table.dtype),
        scratch_shapes=[pltpu.SemaphoreType.DMA],
    )(pos_table[:maxlen])
